# Initial kernel scaffold; baseline (speedup 1.0000x reference)
import jax
import jax.numpy as jnp
from jax.experimental import pallas as pl


def kernel(src, tar):
    raise NotImplementedError("write your pallas kernel here")


# speedup vs baseline: 3444.5278x; 3444.5278x over previous
import functools

import jax
import jax.numpy as jnp
from jax import lax
from jax.experimental import pallas as pl
from jax.experimental.pallas import tpu as pltpu
from jax.experimental.pallas import tpu_sc as plsc

LAMBDA = 0.05
EPS = 1e-12
NBINS = 256
NWORKERS = 32
LANES = 16

N_TOTAL = 64 * 3 * 256 * 256
PER_WORKER = N_TOTAL // NWORKERS
CHUNK = 16384
NCHUNK = PER_WORKER // CHUNK


def _hist_body(tar_hbm, out_hbm, buf, hist, red, sem):
    wid = lax.axis_index("s") * 2 + lax.axis_index("c")
    base = wid * PER_WORKER

    lane = lax.iota(jnp.int32, LANES)
    ones = jnp.full((LANES,), 1.0, jnp.float32)
    zeros = jnp.zeros((LANES,), jnp.float32)

    def zero_body(i, _):
        r = i // (NBINS // LANES)
        c = i % (NBINS // LANES)
        hist[r, pl.ds(c * LANES, LANES)] = zeros
        return ()

    lax.fori_loop(0, LANES * (NBINS // LANES), zero_body, ())

    def chunk_body(k, _):
        pltpu.sync_copy(tar_hbm.at[pl.ds(base + k * CHUNK, CHUNK)], buf)

        def vec_body(i, _):
            v = buf[pl.ds(i * LANES, LANES)]
            idx = v.astype(jnp.int32)
            plsc.addupdate_scatter(hist, [lane, idx], ones)
            return ()

        lax.fori_loop(0, CHUNK // LANES, vec_body, ())
        return ()

    lax.fori_loop(0, NCHUNK, chunk_body, ())

    def red_body(c, _):
        acc = hist[0, pl.ds(c * LANES, LANES)]
        for l in range(1, LANES):
            acc = acc + hist[l, pl.ds(c * LANES, LANES)]
        red[pl.ds(c * LANES, LANES)] = acc
        return ()

    lax.fori_loop(0, NBINS // LANES, red_body, ())
    pltpu.sync_copy(red, out_hbm.at[wid])


_hist_call = functools.partial(
    pl.kernel,
    mesh=plsc.VectorSubcoreMesh(core_axis_name="c", subcore_axis_name="s"),
    out_type=jax.ShapeDtypeStruct((NWORKERS, NBINS), jnp.float32),
    scratch_types=[
        pltpu.VMEM((CHUNK,), jnp.float32),
        pltpu.VMEM((LANES, NBINS), jnp.float32),
        pltpu.VMEM((NBINS,), jnp.float32),
        pltpu.SemaphoreType.DMA,
    ],
    compiler_params=pltpu.CompilerParams(
        use_tc_tiling_on_sc=False, needs_layout_passes=False
    ),
)(_hist_body)


ROWS = N_TOTAL // NBINS
ROW_BLK = 2048
GRID = ROWS // ROW_BLK


def _ssum_body(src_ref, tar_ref, out_ref):
    d = src_ref[...] - tar_ref[...]
    p = jnp.sum(d * d, axis=0, keepdims=True)

    @pl.when(pl.program_id(0) == 0)
    def _init():
        out_ref[...] = jnp.zeros_like(out_ref)

    out_ref[...] += p


_ssum_call = pl.pallas_call(
    _ssum_body,
    grid=(GRID,),
    in_specs=[
        pl.BlockSpec((ROW_BLK, NBINS), lambda i: (i, 0)),
        pl.BlockSpec((ROW_BLK, NBINS), lambda i: (i, 0)),
    ],
    out_specs=pl.BlockSpec((1, NBINS), lambda i: (0, 0)),
    out_shape=jax.ShapeDtypeStruct((1, NBINS), jnp.float32),
)


def _combine_body(cnt_ref, s_ref, out_ref):
    counts = jnp.sum(cnt_ref[...], axis=0, keepdims=True)
    p_y = counts / jnp.float32(N_TOTAL)
    w = LAMBDA / (p_y + EPS)
    loss = jnp.sum(w * s_ref[...]) / jnp.float32(N_TOTAL)
    out_ref[...] = jnp.reshape(loss, (1, 1))


_combine_call = pl.pallas_call(
    _combine_body,
    out_shape=jax.ShapeDtypeStruct((1, 1), jnp.float32),
)


def kernel(src, tar):
    tar_flat = tar.reshape(-1)
    cnt_part = _hist_call(tar_flat)
    s_part = _ssum_call(src.reshape(ROWS, NBINS), tar.reshape(ROWS, NBINS))
    loss = _combine_call(cnt_part, s_part)
    return loss.reshape(())


# speedup vs baseline: 3750.0366x; 1.0887x over previous
import functools

import jax
import jax.numpy as jnp
from jax import lax
from jax.experimental import pallas as pl
from jax.experimental.pallas import tpu as pltpu
from jax.experimental.pallas import tpu_sc as plsc

LAMBDA = 0.05
EPS = 1e-12
NBINS = 256
NWORKERS = 32
LANES = 16

N_TOTAL = 64 * 3 * 256 * 256
NROWS = N_TOTAL // NBINS
ROWS_PER_WORKER = NROWS // NWORKERS
ROWCHUNK = 128
NCHUNK = ROWS_PER_WORKER // ROWCHUNK


def _hist_body(tar_hbm, out_hbm, buf0, buf1, hist, red, sem0, sem1):
    wid = lax.axis_index("s") * 2 + lax.axis_index("c")
    rbase = wid * ROWS_PER_WORKER

    lane = lax.iota(jnp.int32, LANES)
    ones = jnp.full((LANES,), 1.0, jnp.float32)
    zeros = jnp.zeros((LANES,), jnp.float32)

    def _zero(i, _):
        r = i // (NBINS // LANES)
        c = i % (NBINS // LANES)
        hist[r, pl.ds(c * LANES, LANES)] = zeros
        return ()

    lax.fori_loop(0, LANES * (NBINS // LANES), _zero, ())

    def _chunk(k, _):
        pltpu.sync_copy(tar_hbm.at[pl.ds(rbase + k * ROWCHUNK, ROWCHUNK)], buf0)

        def _scan_row(r, _):
            for c in range(NBINS // LANES):
                v = buf0[r, pl.ds(c * LANES, LANES)]
                idx = v.astype(jnp.int32)
                plsc.addupdate_scatter(hist, [lane, idx], ones)
            return ()

        lax.fori_loop(0, ROWCHUNK, _scan_row, ())
        return ()

    lax.fori_loop(0, NCHUNK, _chunk, ())

    def _reduce(c, _):
        acc = hist[0, pl.ds(c * LANES, LANES)]
        for l in range(1, LANES):
            acc = acc + hist[l, pl.ds(c * LANES, LANES)]
        red[pl.ds(c * LANES, LANES)] = acc
        return ()

    lax.fori_loop(0, NBINS // LANES, _reduce, ())

    pltpu.sync_copy(red, out_hbm.at[wid])


_hist_call = functools.partial(
    pl.kernel,
    mesh=plsc.VectorSubcoreMesh(core_axis_name="c", subcore_axis_name="s"),
    out_type=jax.ShapeDtypeStruct((NWORKERS, NBINS), jnp.float32),
    scratch_types=[
        pltpu.VMEM((ROWCHUNK, NBINS), jnp.float32),
        pltpu.VMEM((ROWCHUNK, NBINS), jnp.float32),
        pltpu.VMEM((LANES, NBINS), jnp.float32),
        pltpu.VMEM((NBINS,), jnp.float32),
        pltpu.SemaphoreType.DMA,
        pltpu.SemaphoreType.DMA,
    ],
    compiler_params=pltpu.CompilerParams(
        use_tc_tiling_on_sc=False, needs_layout_passes=False
    ),
)(_hist_body)


ROWS = NROWS
ROW_BLK = 2048
GRID = ROWS // ROW_BLK


def _ssum_body(src_ref, tar_ref, out_ref):
    d = src_ref[...] - tar_ref[...]
    p = jnp.sum(d * d, axis=0, keepdims=True)

    @pl.when(pl.program_id(0) == 0)
    def _init():
        out_ref[...] = jnp.zeros_like(out_ref)

    out_ref[...] += p


_ssum_call = pl.pallas_call(
    _ssum_body,
    grid=(GRID,),
    in_specs=[
        pl.BlockSpec((ROW_BLK, NBINS), lambda i: (i, 0)),
        pl.BlockSpec((ROW_BLK, NBINS), lambda i: (i, 0)),
    ],
    out_specs=pl.BlockSpec((1, NBINS), lambda i: (0, 0)),
    out_shape=jax.ShapeDtypeStruct((1, NBINS), jnp.float32),
)


def _combine_body(cnt_ref, s_ref, out_ref):
    counts = jnp.sum(cnt_ref[...], axis=0, keepdims=True)
    p_y = counts / jnp.float32(N_TOTAL)
    w = LAMBDA / (p_y + EPS)
    loss = jnp.sum(w * s_ref[...]) / jnp.float32(N_TOTAL)
    out_ref[...] = jnp.reshape(loss, (1, 1))


_combine_call = pl.pallas_call(
    _combine_body,
    out_shape=jax.ShapeDtypeStruct((1, 1), jnp.float32),
)


def kernel(src, tar):
    tar2d = tar.reshape(ROWS, NBINS)
    cnt_part = _hist_call(tar2d)
    s_part = _ssum_call(src.reshape(ROWS, NBINS), tar2d)
    loss = _combine_call(cnt_part, s_part)
    return loss.reshape(())


# speedup vs baseline: 3947.8503x; 1.0527x over previous
import functools

import jax
import jax.numpy as jnp
from jax import lax
from jax.experimental import pallas as pl
from jax.experimental.pallas import tpu as pltpu
from jax.experimental.pallas import tpu_sc as plsc

LAMBDA = 0.05
EPS = 1e-12
NBINS = 256
NWORKERS = 32
LANES = 16

N_TOTAL = 64 * 3 * 256 * 256
NROWS = N_TOTAL // NBINS
ROWS_PER_WORKER = NROWS // NWORKERS
ROWCHUNK = 128
NCHUNK = ROWS_PER_WORKER // ROWCHUNK


def _hist_body(tar_hbm, out_hbm, buf0, buf1, hist, red, sem0, sem1):
    wid = lax.axis_index("s") * 2 + lax.axis_index("c")
    rbase = wid * ROWS_PER_WORKER

    lane = lax.iota(jnp.int32, LANES)
    ones = jnp.full((LANES,), 1.0, jnp.float32)
    zeros = jnp.zeros((LANES,), jnp.float32)

    def _zero(i, _):
        r = i // (NBINS // LANES)
        c = i % (NBINS // LANES)
        hist[r, pl.ds(c * LANES, LANES)] = zeros
        return ()

    lax.fori_loop(0, LANES * (NBINS // LANES), _zero, ())

    bufs = (buf0, buf1)
    sems = (sem0, sem1)

    def start(k):
        return pltpu.async_copy(
            tar_hbm.at[pl.ds(rbase + k * ROWCHUNK, ROWCHUNK)], bufs[k % 2], sems[k % 2]
        )

    cp = start(0)
    for k in range(NCHUNK):
        nxt = start(k + 1) if k + 1 < NCHUNK else None
        cp.wait()
        buf = bufs[k % 2]

        def _scan_row(r, _):
            for c in range(NBINS // LANES):
                v = buf[r, pl.ds(c * LANES, LANES)]
                idx = v.astype(jnp.int32)
                plsc.addupdate_scatter(hist, [lane, idx], ones)
            return ()

        lax.fori_loop(0, ROWCHUNK, _scan_row, (), unroll=2)
        cp = nxt

    def _reduce(c, _):
        acc = hist[0, pl.ds(c * LANES, LANES)]
        for l in range(1, LANES):
            acc = acc + hist[l, pl.ds(c * LANES, LANES)]
        red[pl.ds(c * LANES, LANES)] = acc
        return ()

    lax.fori_loop(0, NBINS // LANES, _reduce, ())

    pltpu.sync_copy(red, out_hbm.at[wid])


_hist_call = functools.partial(
    pl.kernel,
    mesh=plsc.VectorSubcoreMesh(core_axis_name="c", subcore_axis_name="s"),
    out_type=jax.ShapeDtypeStruct((NWORKERS, NBINS), jnp.float32),
    scratch_types=[
        pltpu.VMEM((ROWCHUNK, NBINS), jnp.float32),
        pltpu.VMEM((ROWCHUNK, NBINS), jnp.float32),
        pltpu.VMEM((LANES, NBINS), jnp.float32),
        pltpu.VMEM((NBINS,), jnp.float32),
        pltpu.SemaphoreType.DMA,
        pltpu.SemaphoreType.DMA,
    ],
    compiler_params=pltpu.CompilerParams(
        use_tc_tiling_on_sc=False, needs_layout_passes=False
    ),
)(_hist_body)


ROWS = NROWS
ROW_BLK = 2048
GRID = ROWS // ROW_BLK


def _ssum_body(src_ref, tar_ref, out_ref):
    d = src_ref[...] - tar_ref[...]
    p = jnp.sum(d * d, axis=0, keepdims=True)

    @pl.when(pl.program_id(0) == 0)
    def _init():
        out_ref[...] = jnp.zeros_like(out_ref)

    out_ref[...] += p


_ssum_call = pl.pallas_call(
    _ssum_body,
    grid=(GRID,),
    in_specs=[
        pl.BlockSpec((ROW_BLK, NBINS), lambda i: (i, 0)),
        pl.BlockSpec((ROW_BLK, NBINS), lambda i: (i, 0)),
    ],
    out_specs=pl.BlockSpec((1, NBINS), lambda i: (0, 0)),
    out_shape=jax.ShapeDtypeStruct((1, NBINS), jnp.float32),
)


def _combine_body(cnt_ref, s_ref, out_ref):
    counts = jnp.sum(cnt_ref[...], axis=0, keepdims=True)
    p_y = counts / jnp.float32(N_TOTAL)
    w = LAMBDA / (p_y + EPS)
    loss = jnp.sum(w * s_ref[...]) / jnp.float32(N_TOTAL)
    out_ref[...] = jnp.reshape(loss, (1, 1))


_combine_call = pl.pallas_call(
    _combine_body,
    out_shape=jax.ShapeDtypeStruct((1, 1), jnp.float32),
)


def kernel(src, tar):
    tar2d = tar.reshape(ROWS, NBINS)
    cnt_part = _hist_call(tar2d)
    s_part = _ssum_call(src.reshape(ROWS, NBINS), tar2d)
    loss = _combine_call(cnt_part, s_part)
    return loss.reshape(())


# speedup vs baseline: 9203.6606x; 2.3313x over previous
import functools

import jax
import jax.numpy as jnp
from jax import lax
from jax.experimental import pallas as pl
from jax.experimental.pallas import tpu as pltpu
from jax.experimental.pallas import tpu_sc as plsc

LAMBDA = 0.05
EPS = 1e-12
NBINS = 256
NWORKERS = 32
LANES = 16

N_TOTAL = 64 * 3 * 256 * 256
NROWS = N_TOTAL // NBINS
ROWS_PER_WORKER = NROWS // NWORKERS
ROWCHUNK = 128
NCHUNK = ROWS_PER_WORKER // ROWCHUNK


def _hist_body(tar_hbm, out_hbm, buf0, buf1, hist, red, sem0, sem1):
    wid = lax.axis_index("s") * 2 + lax.axis_index("c")
    rbase = wid * ROWS_PER_WORKER

    lane = lax.iota(jnp.int32, LANES)
    ones = jnp.full((LANES,), 1.0, jnp.float32)
    zeros = jnp.zeros((LANES,), jnp.float32)

    def _zero(i, _):
        r = i // (NBINS // LANES)
        c = i % (NBINS // LANES)
        hist[r, pl.ds(c * LANES, LANES)] = zeros
        return ()

    lax.fori_loop(0, LANES * (NBINS // LANES), _zero, ())

    bufs = (buf0, buf1)
    sems = (sem0, sem1)

    def start(k):
        return pltpu.async_copy(
            tar_hbm.at[pl.ds(rbase + k * ROWCHUNK, ROWCHUNK)], bufs[k % 2], sems[k % 2]
        )

    cp = start(0)
    for k in range(NCHUNK):
        nxt = start(k + 1) if k + 1 < NCHUNK else None
        cp.wait()
        buf = bufs[k % 2]

        def _scan_row(r, _):
            vals = [buf[r, pl.ds(c * LANES, LANES)] for c in range(NBINS // LANES)]
            idxs = [v.astype(jnp.int32) for v in vals]
            for idx in idxs:
                plsc.addupdate_scatter(hist, [lane, idx], ones)
            return ()

        lax.fori_loop(0, ROWCHUNK, _scan_row, (), unroll=2)
        cp = nxt

    def _reduce(c, _):
        acc = hist[0, pl.ds(c * LANES, LANES)]
        for l in range(1, LANES):
            acc = acc + hist[l, pl.ds(c * LANES, LANES)]
        red[pl.ds(c * LANES, LANES)] = acc
        return ()

    lax.fori_loop(0, NBINS // LANES, _reduce, ())

    pltpu.sync_copy(red, out_hbm.at[wid])


_hist_call = functools.partial(
    pl.kernel,
    mesh=plsc.VectorSubcoreMesh(core_axis_name="c", subcore_axis_name="s"),
    out_type=jax.ShapeDtypeStruct((NWORKERS, NBINS), jnp.float32),
    scratch_types=[
        pltpu.VMEM((ROWCHUNK, NBINS), jnp.float32),
        pltpu.VMEM((ROWCHUNK, NBINS), jnp.float32),
        pltpu.VMEM((LANES, NBINS), jnp.float32),
        pltpu.VMEM((NBINS,), jnp.float32),
        pltpu.SemaphoreType.DMA,
        pltpu.SemaphoreType.DMA,
    ],
    compiler_params=pltpu.CompilerParams(
        use_tc_tiling_on_sc=False, needs_layout_passes=False
    ),
)(_hist_body)


ROWS = NROWS
ROW_BLK = 2048
GRID = ROWS // ROW_BLK


def _ssum_body(src_ref, tar_ref, out_ref):
    d = src_ref[...] - tar_ref[...]
    p = jnp.sum(d * d, axis=0, keepdims=True)

    @pl.when(pl.program_id(0) == 0)
    def _init():
        out_ref[...] = jnp.zeros_like(out_ref)

    out_ref[...] += p


_ssum_call = pl.pallas_call(
    _ssum_body,
    grid=(GRID,),
    in_specs=[
        pl.BlockSpec((ROW_BLK, NBINS), lambda i: (i, 0)),
        pl.BlockSpec((ROW_BLK, NBINS), lambda i: (i, 0)),
    ],
    out_specs=pl.BlockSpec((1, NBINS), lambda i: (0, 0)),
    out_shape=jax.ShapeDtypeStruct((1, NBINS), jnp.float32),
)


def _combine_body(cnt_ref, s_ref, out_ref):
    counts = jnp.sum(cnt_ref[...], axis=0, keepdims=True)
    p_y = counts / jnp.float32(N_TOTAL)
    w = LAMBDA / (p_y + EPS)
    loss = jnp.sum(w * s_ref[...]) / jnp.float32(N_TOTAL)
    out_ref[...] = jnp.reshape(loss, (1, 1))


_combine_call = pl.pallas_call(
    _combine_body,
    out_shape=jax.ShapeDtypeStruct((1, 1), jnp.float32),
)


def kernel(src, tar):
    tar2d = tar.reshape(ROWS, NBINS)
    cnt_part = _hist_call(tar2d)
    s_part = _ssum_call(src.reshape(ROWS, NBINS), tar2d)
    loss = _combine_call(cnt_part, s_part)
    return loss.reshape(())


# speedup vs baseline: 11348.0663x; 1.2330x over previous
import functools

import jax
import jax.numpy as jnp
from jax import lax
from jax.experimental import pallas as pl
from jax.experimental.pallas import tpu as pltpu
from jax.experimental.pallas import tpu_sc as plsc

LAMBDA = 0.05
EPS = 1e-12
NBINS = 256
NWORKERS = 32
LANES = 16

N_TOTAL = 64 * 3 * 256 * 256
NROWS = N_TOTAL // NBINS
ROWS_PER_WORKER = NROWS // NWORKERS
ROWCHUNK = 128
NCHUNK = ROWS_PER_WORKER // ROWCHUNK


def _hist_body(tar_hbm, out_hbm, buf0, buf1, hist, red, sem0, sem1):
    wid = lax.axis_index("s") * 2 + lax.axis_index("c")
    rbase = wid * ROWS_PER_WORKER

    lane = lax.iota(jnp.int32, LANES)
    ones = jnp.full((LANES,), 1.0, jnp.float32)
    zeros = jnp.zeros((LANES,), jnp.float32)

    def _zero(i, _):
        r = i // (NBINS // LANES)
        c = i % (NBINS // LANES)
        hist[r, pl.ds(c * LANES, LANES)] = zeros
        return ()

    lax.fori_loop(0, LANES * (NBINS // LANES), _zero, ())

    bufs = (buf0, buf1)
    sems = (sem0, sem1)

    def start(k):
        return pltpu.async_copy(
            tar_hbm.at[pl.ds(rbase + k * ROWCHUNK, ROWCHUNK)], bufs[k % 2], sems[k % 2]
        )

    cp = start(0)
    for k in range(NCHUNK):
        nxt = start(k + 1) if k + 1 < NCHUNK else None
        cp.wait()
        buf = bufs[k % 2]

        def _scan_row(r, _):
            vals = [buf[r, pl.ds(c * LANES, LANES)] for c in range(NBINS // LANES)]
            idxs = [v.astype(jnp.int32) for v in vals]
            for idx in idxs:
                plsc.addupdate_scatter(hist, [lane, idx], ones)
            return ()

        lax.fori_loop(0, ROWCHUNK, _scan_row, (), unroll=2)
        cp = nxt

    def _reduce(c, _):
        acc = hist[0, pl.ds(c * LANES, LANES)]
        for l in range(1, LANES):
            acc = acc + hist[l, pl.ds(c * LANES, LANES)]
        red[pl.ds(c * LANES, LANES)] = acc
        return ()

    lax.fori_loop(0, NBINS // LANES, _reduce, ())

    pltpu.sync_copy(red, out_hbm.at[wid])


_hist_call = functools.partial(
    pl.kernel,
    mesh=plsc.VectorSubcoreMesh(core_axis_name="c", subcore_axis_name="s"),
    out_type=jax.ShapeDtypeStruct((NWORKERS, NBINS), jnp.float32),
    scratch_types=[
        pltpu.VMEM((ROWCHUNK, NBINS), jnp.float32),
        pltpu.VMEM((ROWCHUNK, NBINS), jnp.float32),
        pltpu.VMEM((LANES, NBINS), jnp.float32),
        pltpu.VMEM((NBINS,), jnp.float32),
        pltpu.SemaphoreType.DMA,
        pltpu.SemaphoreType.DMA,
    ],
    compiler_params=pltpu.CompilerParams(needs_layout_passes=False),
)(_hist_body)


ROWS = NROWS
ROW_BLK = 2048
GRID = ROWS // ROW_BLK


def _ssum_body(src_ref, tar_ref, out_ref):
    d = src_ref[...] - tar_ref[...]
    p = jnp.sum(d * d, axis=0, keepdims=True)

    @pl.when(pl.program_id(0) == 0)
    def _init():
        out_ref[...] = jnp.zeros_like(out_ref)

    out_ref[...] += p


_ssum_call = pl.pallas_call(
    _ssum_body,
    grid=(GRID,),
    in_specs=[
        pl.BlockSpec((ROW_BLK, NBINS), lambda i: (i, 0)),
        pl.BlockSpec((ROW_BLK, NBINS), lambda i: (i, 0)),
    ],
    out_specs=pl.BlockSpec((1, NBINS), lambda i: (0, 0)),
    out_shape=jax.ShapeDtypeStruct((1, NBINS), jnp.float32),
)


def _combine_body(cnt_ref, s_ref, out_ref):
    counts = jnp.sum(cnt_ref[...], axis=0, keepdims=True)
    p_y = counts / jnp.float32(N_TOTAL)
    w = LAMBDA / (p_y + EPS)
    loss = jnp.sum(w * s_ref[...]) / jnp.float32(N_TOTAL)
    out_ref[...] = jnp.reshape(loss, (1, 1))


_combine_call = pl.pallas_call(
    _combine_body,
    out_shape=jax.ShapeDtypeStruct((1, 1), jnp.float32),
)


def kernel(src, tar):
    tar2d = tar.reshape(ROWS, NBINS)
    cnt_part = _hist_call(tar2d)
    s_part = _ssum_call(src.reshape(ROWS, NBINS), tar2d)
    loss = _combine_call(cnt_part, s_part)
    return loss.reshape(())
